# baseline (device time: 47580 ns/iter reference)
import jax
import jax.numpy as jnp
from jax import lax
from jax.experimental import pallas as pl
from jax.experimental.pallas import tpu as pltpu

N_DEV = 32
V_PER = 4096
N_IDX = 1024
D = 512
B = N_IDX // N_DEV
CHUNK = 1024
RCHUNK = 256
BLOCKS_PER_RC = RCHUNK // B


def kernel(table, idx):
    def body(table_ref, idx_ref, out_ref, gather_buf,
             send1, recv1, send2, recv2):
        me = lax.axis_index("i")

        barrier_sem = pltpu.get_barrier_semaphore()
        for d in range(1, N_DEV):
            pl.semaphore_signal(
                barrier_sem, inc=1,
                device_id=((me + d) % N_DEV,),
                device_id_type=pl.DeviceIdType.MESH,
            )
        pl.semaphore_wait(barrier_sem, N_DEV - 1)

        p1 = []
        for p in range(N_DEV):
            d = (p - me) % N_DEV
            p1.append(pltpu.make_async_remote_copy(
                src_ref=out_ref.at[pl.ds(p * B, B), :],
                dst_ref=gather_buf.at[pl.ds(d * B, B), :],
                send_sem=send1.at[p],
                recv_sem=recv1.at[d],
                device_id=(p,),
                device_id_type=pl.DeviceIdType.MESH,
            ))

        local = idx_ref[:] - me * V_PER
        for rc in range(N_IDX // RCHUNK):
            rows = local[rc * RCHUNK:(rc + 1) * RCHUNK].reshape(RCHUNK, 1)
            acc = jnp.zeros((RCHUNK, D), jnp.float32)
            for j in range(V_PER // CHUNK):
                cols = (lax.broadcasted_iota(jnp.int32, (RCHUNK, CHUNK), 1)
                        + j * CHUNK)
                onehot = (cols == rows).astype(jnp.bfloat16)
                t_chunk = table_ref[j * CHUNK:(j + 1) * CHUNK, :].astype(
                    jnp.bfloat16)
                acc = acc + jnp.dot(onehot, t_chunk,
                                    preferred_element_type=jnp.float32)
            out_ref[pl.ds(rc * RCHUNK, RCHUNK), :] = acc.astype(jnp.bfloat16)
            for p in range(rc * BLOCKS_PER_RC, (rc + 1) * BLOCKS_PER_RC):
                @pl.when(p != me)
                def _(p=p):
                    p1[p].start()

        gather_buf[pl.ds(0, B), :] = out_ref[pl.ds(me * B, B), :]
        for d in range(1, N_DEV):
            dummy = pltpu.make_async_remote_copy(
                src_ref=gather_buf.at[pl.ds(d * B, B), :],
                dst_ref=gather_buf.at[pl.ds(d * B, B), :],
                send_sem=send2.at[d],
                recv_sem=recv1.at[d],
                device_id=(0,),
                device_id_type=pl.DeviceIdType.MESH,
            )
            dummy.wait_recv()
        blk = gather_buf[pl.ds(0, B), :]
        for d in range(1, N_DEV):
            blk = blk + gather_buf[d * B:(d + 1) * B, :]
        out_ref[pl.ds(me * B, B), :] = blk

        p2 = []
        for d in range(1, N_DEV):
            rdma = pltpu.make_async_remote_copy(
                src_ref=out_ref.at[pl.ds(me * B, B), :],
                dst_ref=out_ref.at[pl.ds(me * B, B), :],
                send_sem=send2.at[d],
                recv_sem=recv2.at[d],
                device_id=((me + d) % N_DEV,),
                device_id_type=pl.DeviceIdType.MESH,
            )
            rdma.start()
            p2.append(rdma)

        for p in range(N_DEV):
            @pl.when(p != me)
            def _(p=p):
                p1[p].wait_send()
        for d in range(1, N_DEV):
            p2[d - 1].wait()

    return pl.pallas_call(
        body,
        out_shape=jax.ShapeDtypeStruct((N_IDX, D), jnp.bfloat16),
        in_specs=[
            pl.BlockSpec(memory_space=pltpu.VMEM),
            pl.BlockSpec(memory_space=pltpu.VMEM),
        ],
        out_specs=pl.BlockSpec(memory_space=pltpu.VMEM),
        scratch_shapes=[
            pltpu.VMEM((N_DEV * B, D), jnp.bfloat16),
            pltpu.SemaphoreType.DMA((N_DEV,)),
            pltpu.SemaphoreType.DMA((N_DEV,)),
            pltpu.SemaphoreType.DMA((N_DEV,)),
            pltpu.SemaphoreType.DMA((N_DEV,)),
        ],
        compiler_params=pltpu.CompilerParams(collective_id=0),
    )(table, idx)


# device time: 41179 ns/iter; 1.1554x vs baseline; 1.1554x over previous
import jax
import jax.numpy as jnp
from jax import lax
from jax.experimental import pallas as pl
from jax.experimental.pallas import tpu as pltpu

N_DEV = 32
V_PER = 4096
N_IDX = 1024
D = 512
B = N_IDX // N_DEV
CHUNK = 2048


def kernel(table, idx):
    def body(table_ref, idx_ref, out_ref, gather_buf,
             send1, recv1, send2, recv2):
        me = lax.axis_index("i")

        barrier_sem = pltpu.get_barrier_semaphore()
        for d in range(1, N_DEV):
            pl.semaphore_signal(
                barrier_sem, inc=1,
                device_id=((me + d) % N_DEV,),
                device_id_type=pl.DeviceIdType.MESH,
            )

        local = idx_ref[:] - me * V_PER
        local2d = local.reshape(N_IDX, 1)
        acc = jnp.zeros((N_IDX, D), jnp.float32)
        for j in range(V_PER // CHUNK):
            cols = lax.broadcasted_iota(jnp.int32, (N_IDX, CHUNK), 1) + j * CHUNK
            onehot = (cols == local2d).astype(jnp.bfloat16)
            t_chunk = table_ref[j * CHUNK:(j + 1) * CHUNK, :].astype(jnp.bfloat16)
            acc = acc + jnp.dot(onehot, t_chunk,
                                preferred_element_type=jnp.float32)
        out_ref[...] = acc.astype(jnp.bfloat16)

        pl.semaphore_wait(barrier_sem, N_DEV - 1)

        p1 = []
        for d in range(1, N_DEV):
            p = (me + d) % N_DEV
            rdma = pltpu.make_async_remote_copy(
                src_ref=out_ref.at[pl.ds(p * B, B), :],
                dst_ref=gather_buf.at[pl.ds(d * B, B), :],
                send_sem=send1.at[d],
                recv_sem=recv1.at[d],
                device_id=(p,),
                device_id_type=pl.DeviceIdType.MESH,
            )
            rdma.start()
            p1.append(rdma)

        gather_buf[pl.ds(0, B), :] = out_ref[pl.ds(me * B, B), :]
        for d in range(1, N_DEV):
            p1[d - 1].wait_recv()
        blk = gather_buf[pl.ds(0, B), :]
        for d in range(1, N_DEV):
            blk = blk + gather_buf[d * B:(d + 1) * B, :]
        out_ref[pl.ds(me * B, B), :] = blk

        p2 = []
        for d in range(1, N_DEV):
            rdma = pltpu.make_async_remote_copy(
                src_ref=out_ref.at[pl.ds(me * B, B), :],
                dst_ref=out_ref.at[pl.ds(me * B, B), :],
                send_sem=send2.at[d],
                recv_sem=recv2.at[d],
                device_id=((me + d) % N_DEV,),
                device_id_type=pl.DeviceIdType.MESH,
            )
            rdma.start()
            p2.append(rdma)

        for d in range(1, N_DEV):
            p1[d - 1].wait_send()
        for d in range(1, N_DEV):
            p2[d - 1].wait()

    return pl.pallas_call(
        body,
        out_shape=jax.ShapeDtypeStruct((N_IDX, D), jnp.bfloat16),
        in_specs=[
            pl.BlockSpec(memory_space=pltpu.VMEM),
            pl.BlockSpec(memory_space=pltpu.VMEM),
        ],
        out_specs=pl.BlockSpec(memory_space=pltpu.VMEM),
        scratch_shapes=[
            pltpu.VMEM((N_DEV * B, D), jnp.bfloat16),
            pltpu.SemaphoreType.DMA((N_DEV,)),
            pltpu.SemaphoreType.DMA((N_DEV,)),
            pltpu.SemaphoreType.DMA((N_DEV,)),
            pltpu.SemaphoreType.DMA((N_DEV,)),
        ],
        compiler_params=pltpu.CompilerParams(collective_id=0),
    )(table, idx)
